# baseline (speedup 1.0000x reference)
import functools

import jax
import jax.numpy as jnp
from jax.experimental import pallas as pl
from jax.experimental.pallas import tpu as pltpu


def _rnn_packed_kernel(x_ref, wih_ref, whh_ref, b_ref, w1_ref, b1_ref,
                       w2_ref, s_ref, b2_ref, out_ref, h_ref,
                       *, t_tile, n_chunks):
    ti = pl.program_id(0)

    @pl.when(ti == 0)
    def _():
        h_ref[...] = jnp.zeros_like(h_ref)

    rows = h_ref.shape[0]
    rc = rows // n_chunks

    gh = h_ref.shape[1]
    pre = (jnp.dot(x_ref[...], wih_ref[...],
                   preferred_element_type=jnp.float32)
           + b_ref[...])

    whh = whh_ref[...]
    hs = [h_ref[c * rc:(c + 1) * rc, :] for c in range(n_chunks)]
    for t in range(t_tile):
        base = (t // 2) * rows
        lo = (t % 2) * gh
        for c in range(n_chunks):
            p = pre[base + c * rc:base + (c + 1) * rc, lo:lo + gh]
            hs[c] = jnp.tanh(p + jnp.dot(hs[c], whh,
                                         preferred_element_type=jnp.float32))
    for c in range(n_chunks):
        h_ref[c * rc:(c + 1) * rc, :] = hs[c]

    @pl.when(ti == pl.num_programs(0) - 1)
    def _():
        h = jnp.concatenate(hs, axis=0)
        z1 = (jnp.dot(h, w1_ref[...], preferred_element_type=jnp.float32)
              + b1_ref[...])
        z1 = jnp.maximum(z1, 0.0)
        zz = z1 * w2_ref[...]
        out = (jnp.dot(zz, s_ref[...], preferred_element_type=jnp.float32)
               + b2_ref[0, 0])
        out_ref[...] = out


def _pick_t_tile(T, cap=32):
    best = 1
    for cand in range(1, min(T, cap) + 1):
        if T % cand == 0:
            best = cand
    return best


@jax.jit
def kernel(x, w_ih, w_hh, b_ih, b_hh, w_fc1, b_fc1, w_fc2, b_fc2):
    B, T, I = x.shape
    H = w_hh.shape[0]
    F = w_fc1.shape[1]

    G = 4
    assert B % G == 0
    R = B // G
    assert R % 8 == 0
    t_tile = _pick_t_tile(T)
    nt = T // t_tile
    n_chunks = 2 if R % 16 == 0 else 1

    f32 = jnp.float32
    bf16 = jnp.bfloat16
    x = x.astype(bf16)

    x_rows = (x.reshape(G, R, T // 2, 2, I)
              .transpose(2, 1, 3, 0, 4)
              .reshape(T // 2 * R, 2 * G * I))

    eye = jnp.eye(G, dtype=f32)
    wih_bd = jnp.kron(jnp.eye(2, dtype=f32),
                      jnp.kron(eye, w_ih.astype(f32))).astype(bf16)
    whh_bd = jnp.kron(eye, w_hh.astype(f32))
    b_pk = jnp.tile((b_ih + b_hh).astype(f32), (1, 2 * G))
    w1_bd = jnp.kron(eye, w_fc1.astype(f32))
    b1_pk = jnp.tile(b_fc1.astype(f32), (1, G))
    w2_pk = jnp.tile(w_fc2.astype(f32).reshape(1, F), (1, G))
    seg = jnp.kron(eye, jnp.ones((F, 1), f32))
    b2 = jnp.asarray(b_fc2, f32).reshape(1, 1)

    cost = pl.CostEstimate(
        flops=2 * T * B * (I * H + H * H) + 2 * B * (H * F + F),
        transcendentals=T * B * H,
        bytes_accessed=4 * (T * B * I + B),
    )

    grid_spec = pltpu.PrefetchScalarGridSpec(
        num_scalar_prefetch=0,
        grid=(nt,),
        in_specs=[
            pl.BlockSpec((t_tile // 2 * R, 2 * G * I), lambda t: (t, 0)),
            pl.BlockSpec((2 * G * I, 2 * G * H), lambda t: (0, 0)),
            pl.BlockSpec((G * H, G * H), lambda t: (0, 0)),
            pl.BlockSpec((1, 2 * G * H), lambda t: (0, 0)),
            pl.BlockSpec((G * H, G * F), lambda t: (0, 0)),
            pl.BlockSpec((1, G * F), lambda t: (0, 0)),
            pl.BlockSpec((1, G * F), lambda t: (0, 0)),
            pl.BlockSpec((G * F, G), lambda t: (0, 0)),
            pl.BlockSpec((1, 1), lambda t: (0, 0)),
        ],
        out_specs=pl.BlockSpec((R, G), lambda t: (0, 0)),
        scratch_shapes=[pltpu.VMEM((R, G * H), f32)],
    )

    out = pl.pallas_call(
        functools.partial(_rnn_packed_kernel, t_tile=t_tile,
                          n_chunks=n_chunks),
        out_shape=jax.ShapeDtypeStruct((R, G), f32),
        grid_spec=grid_spec,
        compiler_params=pltpu.CompilerParams(
            dimension_semantics=("arbitrary",),
            vmem_limit_bytes=48 * 1024 * 1024,
        ),
        cost_estimate=cost,
    )(x_rows, wih_bd, whh_bd, b_pk, w1_bd, b1_pk, w2_pk, seg, b2)

    return out.transpose(1, 0).reshape(B)


# speedup vs baseline: 1.1793x; 1.1793x over previous
import functools

import jax
import jax.numpy as jnp
from jax.experimental import pallas as pl
from jax.experimental.pallas import tpu as pltpu


def _rnn_packed_kernel(x_ref, wih_ref, whh_ref, b_ref, w1_ref, b1_ref,
                       w2_ref, s_ref, b2_ref, out_ref, h_ref,
                       *, t_tile, n_chunks):
    ti = pl.program_id(0)

    @pl.when(ti == 0)
    def _():
        h_ref[...] = jnp.zeros_like(h_ref)

    rows = h_ref.shape[0]
    rc = rows // n_chunks

    pre = (jnp.dot(x_ref[...], wih_ref[...],
                   preferred_element_type=jnp.float32)
           + b_ref[...])

    whh = whh_ref[...]
    hs = [h_ref[c * rc:(c + 1) * rc, :] for c in range(n_chunks)]
    for t in range(t_tile):
        base = t * rows
        for c in range(n_chunks):
            p = pre[base + c * rc:base + (c + 1) * rc, :]
            hs[c] = jnp.tanh(p + jnp.dot(hs[c], whh,
                                         preferred_element_type=jnp.float32))
    for c in range(n_chunks):
        h_ref[c * rc:(c + 1) * rc, :] = hs[c]

    @pl.when(ti == pl.num_programs(0) - 1)
    def _():
        h = jnp.concatenate(hs, axis=0)
        z1 = (jnp.dot(h, w1_ref[...], preferred_element_type=jnp.float32)
              + b1_ref[...])
        z1 = jnp.maximum(z1, 0.0)
        zz = z1 * w2_ref[...]
        out = (jnp.dot(zz, s_ref[...], preferred_element_type=jnp.float32)
               + b2_ref[0, 0])
        out_ref[...] = out


def _pick_t_tile(T, cap=16):
    best = 1
    for cand in range(1, min(T, cap) + 1):
        if T % cand == 0:
            best = cand
    return best


@jax.jit
def kernel(x, w_ih, w_hh, b_ih, b_hh, w_fc1, b_fc1, w_fc2, b_fc2):
    B, T, I = x.shape
    H = w_hh.shape[0]
    F = w_fc1.shape[1]

    G = 4
    assert B % G == 0
    R = B // G
    assert R % 8 == 0
    t_tile = _pick_t_tile(T)
    nt = T // t_tile
    n_chunks = 2 if R % 16 == 0 else 1

    f32 = jnp.float32
    bf16 = jnp.bfloat16
    x = x.astype(bf16)

    x_rows = (x.reshape(G, R, T, I)
              .transpose(2, 1, 0, 3)
              .reshape(T * R, G * I))

    eye = jnp.eye(G, dtype=f32)
    wih_bd = jnp.kron(eye, w_ih.astype(f32)).astype(bf16)
    whh_bd = jnp.kron(eye, w_hh.astype(f32))
    b_pk = jnp.tile((b_ih + b_hh).astype(f32), (1, G))
    w1_bd = jnp.kron(eye, w_fc1.astype(f32))
    b1_pk = jnp.tile(b_fc1.astype(f32), (1, G))
    w2_pk = jnp.tile(w_fc2.astype(f32).reshape(1, F), (1, G))
    seg = jnp.kron(eye, jnp.ones((F, 1), f32))
    b2 = jnp.asarray(b_fc2, f32).reshape(1, 1)

    cost = pl.CostEstimate(
        flops=2 * T * B * (I * H + H * H) + 2 * B * (H * F + F),
        transcendentals=T * B * H,
        bytes_accessed=4 * (T * B * I + B),
    )

    grid_spec = pltpu.PrefetchScalarGridSpec(
        num_scalar_prefetch=0,
        grid=(nt,),
        in_specs=[
            pl.BlockSpec((t_tile * R, G * I), lambda t: (t, 0)),
            pl.BlockSpec((G * I, G * H), lambda t: (0, 0)),
            pl.BlockSpec((G * H, G * H), lambda t: (0, 0)),
            pl.BlockSpec((1, G * H), lambda t: (0, 0)),
            pl.BlockSpec((G * H, G * F), lambda t: (0, 0)),
            pl.BlockSpec((1, G * F), lambda t: (0, 0)),
            pl.BlockSpec((1, G * F), lambda t: (0, 0)),
            pl.BlockSpec((G * F, G), lambda t: (0, 0)),
            pl.BlockSpec((1, 1), lambda t: (0, 0)),
        ],
        out_specs=pl.BlockSpec((R, G), lambda t: (0, 0)),
        scratch_shapes=[pltpu.VMEM((R, G * H), f32)],
    )

    out = pl.pallas_call(
        functools.partial(_rnn_packed_kernel, t_tile=t_tile,
                          n_chunks=n_chunks),
        out_shape=jax.ShapeDtypeStruct((R, G), f32),
        grid_spec=grid_spec,
        compiler_params=pltpu.CompilerParams(
            dimension_semantics=("arbitrary",),
            vmem_limit_bytes=48 * 1024 * 1024,
        ),
        cost_estimate=cost,
    )(x_rows, wih_bd, whh_bd, b_pk, w1_bd, b1_pk, w2_pk, seg, b2)

    return out.transpose(1, 0).reshape(B)


# speedup vs baseline: 1.2020x; 1.0193x over previous
import functools

import jax
import jax.numpy as jnp
from jax.experimental import pallas as pl
from jax.experimental.pallas import tpu as pltpu


def _rnn_packed_kernel(x_ref, wih_ref, whh_ref, b_ref, w1_ref, b1_ref,
                       w2_ref, s_ref, b2_ref, out_ref, h_ref,
                       *, t_tile, n_chunks):
    ti = pl.program_id(0)

    @pl.when(ti == 0)
    def _():
        h_ref[...] = jnp.zeros_like(h_ref)

    rows = h_ref.shape[0]
    rc = rows // n_chunks

    pre = (jnp.dot(x_ref[...], wih_ref[...],
                   preferred_element_type=jnp.float32)
           + b_ref[...])

    whh = whh_ref[...]
    hs = [h_ref[c * rc:(c + 1) * rc, :] for c in range(n_chunks)]
    for t in range(t_tile):
        base = t * rows
        for c in range(n_chunks):
            p = pre[base + c * rc:base + (c + 1) * rc, :]
            hs[c] = jnp.tanh(p + jnp.dot(hs[c], whh,
                                         preferred_element_type=jnp.float32))
    for c in range(n_chunks):
        h_ref[c * rc:(c + 1) * rc, :] = hs[c]

    @pl.when(ti == pl.num_programs(0) - 1)
    def _():
        h = jnp.concatenate(hs, axis=0)
        z1 = (jnp.dot(h, w1_ref[...], preferred_element_type=jnp.float32)
              + b1_ref[...])
        z1 = jnp.maximum(z1, 0.0)
        zz = z1 * w2_ref[...]
        out = (jnp.dot(zz, s_ref[...], preferred_element_type=jnp.float32)
               + b2_ref[0, 0])
        out_ref[...] = out


def _pick_t_tile(T, cap=32):
    best = 1
    for cand in range(1, min(T, cap) + 1):
        if T % cand == 0:
            best = cand
    return best


@jax.jit
def kernel(x, w_ih, w_hh, b_ih, b_hh, w_fc1, b_fc1, w_fc2, b_fc2):
    B, T, I = x.shape
    H = w_hh.shape[0]
    F = w_fc1.shape[1]

    G = 4
    assert B % G == 0
    R = B // G
    assert R % 8 == 0
    t_tile = _pick_t_tile(T)
    nt = T // t_tile
    n_chunks = 2 if R % 16 == 0 else 1

    f32 = jnp.float32
    bf16 = jnp.bfloat16
    x = x.astype(bf16)

    x_rows = (x.reshape(G, R, T, I)
              .transpose(2, 1, 0, 3)
              .reshape(T * R, G * I))

    eye = jnp.eye(G, dtype=f32)
    wih_bd = jnp.kron(eye, w_ih.astype(f32)).astype(bf16)
    whh_bd = jnp.kron(eye, w_hh.astype(f32))
    b_pk = jnp.tile((b_ih + b_hh).astype(f32), (1, G))
    w1_bd = jnp.kron(eye, w_fc1.astype(f32))
    b1_pk = jnp.tile(b_fc1.astype(f32), (1, G))
    w2_pk = jnp.tile(w_fc2.astype(f32).reshape(1, F), (1, G))
    seg = jnp.kron(eye, jnp.ones((F, 1), f32))
    b2 = jnp.asarray(b_fc2, f32).reshape(1, 1)

    cost = pl.CostEstimate(
        flops=2 * T * B * (I * H + H * H) + 2 * B * (H * F + F),
        transcendentals=T * B * H,
        bytes_accessed=4 * (T * B * I + B),
    )

    grid_spec = pltpu.PrefetchScalarGridSpec(
        num_scalar_prefetch=0,
        grid=(nt,),
        in_specs=[
            pl.BlockSpec((t_tile * R, G * I), lambda t: (t, 0)),
            pl.BlockSpec((G * I, G * H), lambda t: (0, 0)),
            pl.BlockSpec((G * H, G * H), lambda t: (0, 0)),
            pl.BlockSpec((1, G * H), lambda t: (0, 0)),
            pl.BlockSpec((G * H, G * F), lambda t: (0, 0)),
            pl.BlockSpec((1, G * F), lambda t: (0, 0)),
            pl.BlockSpec((1, G * F), lambda t: (0, 0)),
            pl.BlockSpec((G * F, G), lambda t: (0, 0)),
            pl.BlockSpec((1, 1), lambda t: (0, 0)),
        ],
        out_specs=pl.BlockSpec((R, G), lambda t: (0, 0)),
        scratch_shapes=[pltpu.VMEM((R, G * H), f32)],
    )

    out = pl.pallas_call(
        functools.partial(_rnn_packed_kernel, t_tile=t_tile,
                          n_chunks=n_chunks),
        out_shape=jax.ShapeDtypeStruct((R, G), f32),
        grid_spec=grid_spec,
        compiler_params=pltpu.CompilerParams(
            dimension_semantics=("arbitrary",),
            vmem_limit_bytes=48 * 1024 * 1024,
        ),
        cost_estimate=cost,
    )(x_rows, wih_bd, whh_bd, b_pk, w1_bd, b1_pk, w2_pk, seg, b2)

    return out.transpose(1, 0).reshape(B)
